# baseline (speedup 1.0000x reference)
import functools

import jax
import jax.numpy as jnp
from jax import lax
from jax.experimental import pallas as pl
from jax.experimental.pallas import tpu as pltpu
from jax.experimental.pallas import tpu_sc as plsc

N_FACTORS = 32
BATCH = 16384

NC = 2
NS = 16
L = 16
NW = NC * NS
BPW = BATCH // NW
ICH = 128
NCH = BPW // ICH
NBLK = BPW // L

_mesh = plsc.VectorSubcoreMesh(
    core_axis_name="c", subcore_axis_name="s", num_cores=NC, num_subcores=NS
)


@functools.partial(
    pl.kernel,
    out_type=jax.ShapeDtypeStruct((BATCH,), jnp.float32),
    mesh=_mesh,
    compiler_params=pltpu.CompilerParams(use_tc_tiling_on_sc=False),
    scratch_types=[
        pltpu.VMEM((NCH, ICH), jnp.int32),
        pltpu.VMEM((NCH, ICH), jnp.int32),
        pltpu.VMEM((BPW, N_FACTORS), jnp.float32),
        pltpu.VMEM((BPW, N_FACTORS), jnp.float32),
        pltpu.VMEM((BPW,), jnp.float32),
        pltpu.VMEM((BPW,), jnp.float32),
        pltpu.VMEM((BPW,), jnp.float32),
        pltpu.SemaphoreType.DMA,
    ],
)
def _mf_sc_kernel(user_hbm, item_hbm, uf_hbm, itf_hbm, ub_hbm, ib_hbm,
                  out_hbm, uidx_v, iidx_v, uf_v, itf_v, ub_v, ib_v,
                  out_v, sem):
    wid = lax.axis_index("s") * NC + lax.axis_index("c")
    base = wid * BPW

    for j in range(NCH):
        pltpu.sync_copy(user_hbm.at[pl.ds(base + j * ICH, ICH)], uidx_v.at[j])
        pltpu.sync_copy(item_hbm.at[pl.ds(base + j * ICH, ICH)], iidx_v.at[j])

    copies = []
    for j in range(NCH):
        sl = pl.ds(j * ICH, ICH)
        copies.append(pltpu.async_copy(uf_hbm.at[uidx_v.at[j]], uf_v.at[sl], sem))
        copies.append(pltpu.async_copy(itf_hbm.at[iidx_v.at[j]], itf_v.at[sl], sem))
        copies.append(pltpu.async_copy(ub_hbm.at[uidx_v.at[j]], ub_v.at[sl], sem))
        copies.append(pltpu.async_copy(ib_hbm.at[iidx_v.at[j]], ib_v.at[sl], sem))
    for cp in copies:
        cp.wait()

    lanes = lax.broadcasted_iota(jnp.int32, (L,), 0)

    def block(i, carry):
        b0 = i * L
        partials = []
        for e in range(L):
            u_lo = uf_v[b0 + e, pl.ds(0, L)]
            u_hi = uf_v[b0 + e, pl.ds(L, L)]
            v_lo = itf_v[b0 + e, pl.ds(0, L)]
            v_hi = itf_v[b0 + e, pl.ds(L, L)]
            partials.append(u_lo * v_lo + u_hi * v_hi)
        for s in (1, 2, 4, 8):
            cond = (lanes & s) == 0
            nxt = []
            for j in range(0, len(partials), 2):
                a, c = partials[j], partials[j + 1]
                q = jnp.where(cond, a, c)
                t = jnp.where(cond, c, a)
                nxt.append(q + jnp.take(t, lanes ^ s))
            partials = nxt
        out_v[pl.ds(b0, L)] = (partials[0] + ub_v[pl.ds(b0, L)]
                               + ib_v[pl.ds(b0, L)])
        return carry

    lax.fori_loop(0, NBLK, block, 0)

    pltpu.sync_copy(out_v, out_hbm.at[pl.ds(base, BPW)])


def kernel(user, item, user_factors, item_factors, user_biases, item_biases):
    user = user.astype(jnp.int32)
    item = item.astype(jnp.int32)
    return _mf_sc_kernel(user, item, user_factors, item_factors,
                         user_biases.reshape(-1), item_biases.reshape(-1))
